# Initial kernel scaffold; baseline (speedup 1.0000x reference)
import jax
import jax.numpy as jnp
from jax.experimental import pallas as pl


def kernel(mri_conv1_w, mri_bn1_scale, mri_bn1_shift, mri_l1_conv1_w, mri_l1_bn1_scale, mri_l1_bn1_shift, mri_l1_conv2_w, mri_l1_bn2_scale, mri_l1_bn2_shift, mri_l2_conv1_w, mri_l2_bn1_scale, mri_l2_bn1_shift, mri_l2_conv2_w, mri_l2_bn2_scale, mri_l2_bn2_shift, mri_l2_down_w, mri_l2_down_scale, mri_l2_down_shift, mri_l3_conv1_w, mri_l3_bn1_scale, mri_l3_bn1_shift, mri_l3_conv2_w, mri_l3_bn2_scale, mri_l3_bn2_shift, mri_l3_down_w, mri_l3_down_scale, mri_l3_down_shift, mri_l4_conv1_w, mri_l4_bn1_scale, mri_l4_bn1_shift, mri_l4_conv2_w, mri_l4_bn2_scale, mri_l4_bn2_shift, mri_l4_down_w, mri_l4_down_scale, mri_l4_down_shift, pet_conv1_w, pet_bn1_scale, pet_bn1_shift, pet_l1_conv1_w, pet_l1_bn1_scale, pet_l1_bn1_shift, pet_l1_conv2_w, pet_l1_bn2_scale, pet_l1_bn2_shift, pet_l2_conv1_w, pet_l2_bn1_scale, pet_l2_bn1_shift, pet_l2_conv2_w, pet_l2_bn2_scale, pet_l2_bn2_shift, pet_l2_down_w, pet_l2_down_scale, pet_l2_down_shift, pet_l3_conv1_w, pet_l3_bn1_scale, pet_l3_bn1_shift, pet_l3_conv2_w, pet_l3_bn2_scale, pet_l3_bn2_shift, pet_l3_down_w, pet_l3_down_scale, pet_l3_down_shift, pet_l4_conv1_w, pet_l4_bn1_scale, pet_l4_bn1_shift, pet_l4_conv2_w, pet_l4_bn2_scale, pet_l4_bn2_shift, pet_l4_down_w, pet_l4_down_scale, pet_l4_down_shift, mri_proj_w, mri_proj_shift, pet_proj_w, pet_proj_shift, mri_proj_scale, pet_proj_scale, fc_w, fc_scale, fc_shift, t, mri, pet):
    raise NotImplementedError("write your pallas kernel here")


# speedup vs baseline: 6.6064x; 6.6064x over previous
import functools
import math

import jax
import jax.numpy as jnp
from jax.experimental import pallas as pl
from jax.experimental.pallas import tpu as pltpu

_VMEM_LIMIT = 60 * 1024 * 1024


def _gemm_body(a_ref, w_ref, s_ref, b_ref, o_ref, *, relu):
    acc = jnp.dot(a_ref[0], w_ref[0], preferred_element_type=jnp.float32)
    out = acc * s_ref[0] + b_ref[0]
    if relu:
        out = jnp.maximum(out, 0.0)
    o_ref[0] = out.astype(o_ref.dtype)


def _gemm_res_body(a_ref, w_ref, s_ref, b_ref, r_ref, o_ref, *, relu):
    acc = jnp.dot(a_ref[0], w_ref[0], preferred_element_type=jnp.float32)
    out = acc * s_ref[0] + b_ref[0] + r_ref[0].astype(jnp.float32)
    if relu:
        out = jnp.maximum(out, 0.0)
    o_ref[0] = out.astype(o_ref.dtype)


def _pick_tm(M, K):
    budget = 6 * 1024 * 1024
    tm = budget // (K * 2)
    if tm >= 512:
        tm -= tm % 512
    else:
        tm = 512
    return min(M, min(tm, 4096))


def _gemm_bn(a, w, scale, shift, *, relu, res=None, out_dtype=jnp.bfloat16):
    S, M, K = a.shape
    N = w.shape[-1]
    TM = _pick_tm(M, K)
    grid = (S, pl.cdiv(M, TM))

    in_specs = [
        pl.BlockSpec((1, TM, K), lambda s, m: (s, m, 0)),
        pl.BlockSpec((1, K, N), lambda s, m: (s, 0, 0)),
        pl.BlockSpec((1, 1, N), lambda s, m: (s, 0, 0)),
        pl.BlockSpec((1, 1, N), lambda s, m: (s, 0, 0)),
    ]
    args = [a, w, scale, shift]
    if res is None:
        body = functools.partial(_gemm_body, relu=relu)
    else:
        body = functools.partial(_gemm_res_body, relu=relu)
        in_specs.append(pl.BlockSpec((1, TM, N), lambda s, m: (s, m, 0)))
        args.append(res)

    return pl.pallas_call(
        body,
        out_shape=jax.ShapeDtypeStruct((S, M, N), out_dtype),
        grid=grid,
        in_specs=in_specs,
        out_specs=pl.BlockSpec((1, TM, N), lambda s, m: (s, m, 0)),
        compiler_params=pltpu.CompilerParams(
            dimension_semantics=("parallel", "parallel"),
            vmem_limit_bytes=_VMEM_LIMIT),
    )(*args)


def _im2col(x, ksize, stride, padding):
    kd, kh, kw = ksize
    if padding > 0:
        x = jnp.pad(x, ((0, 0), (0, 0), (padding, padding), (padding, padding),
                        (padding, padding), (0, 0)))
    S, B, Dp, Hp, Wp, C = x.shape
    Do = (Dp - kd) // stride + 1
    Ho = (Hp - kh) // stride + 1
    Wo = (Wp - kw) // stride + 1
    cols = []
    for i in range(kd):
        for j in range(kh):
            for l in range(kw):
                cols.append(x[:, :, i:i + stride * (Do - 1) + 1:stride,
                              j:j + stride * (Ho - 1) + 1:stride,
                              l:l + stride * (Wo - 1) + 1:stride, :])
    a = cols[0] if len(cols) == 1 else jnp.concatenate(cols, axis=-1)
    return a.reshape(S, B * Do * Ho * Wo, kd * kh * kw * C), (B, Do, Ho, Wo)


def _conv_bn_act(x, w, scale, shift, *, ksize, stride, padding, relu, res=None):
    a, (B, Do, Ho, Wo) = _im2col(x, ksize, stride, padding)
    S = x.shape[0]
    N = w.shape[-1]
    r2 = None if res is None else res.reshape(S, -1, N)
    out = _gemm_bn(a, w, scale, shift, relu=relu, res=r2)
    return out.reshape(S, B, Do, Ho, Wo, N)


def _axis_pool(x, axis):
    L = x.shape[axis]
    Lo = L // 2
    shp = x.shape[:axis] + (Lo, 2) + x.shape[axis + 1:]
    xr = x.reshape(shp)
    even = jax.lax.index_in_dim(xr, 0, axis + 1, keepdims=False)
    odd = jax.lax.index_in_dim(xr, 1, axis + 1, keepdims=False)
    pairmax = jnp.maximum(even, odd)
    neg = jnp.finfo(x.dtype).min
    head = jnp.full(odd.shape[:axis] + (1,) + odd.shape[axis + 1:], neg, x.dtype)
    prev_odd = jnp.concatenate(
        [head, jax.lax.slice_in_dim(odd, 0, Lo - 1, axis=axis)], axis=axis)
    return jnp.maximum(pairmax, prev_odd)


def _maxpool_body(x_ref, o_ref):
    x = x_ref[0]
    x = _axis_pool(x, 2)
    x = _axis_pool(x, 1)
    x = _axis_pool(x, 0)
    o_ref[0] = x


def _maxpool3d(x):
    S, B, D, H, W, C = x.shape
    xf = x.reshape(S * B, D, H, W, C)
    out = pl.pallas_call(
        _maxpool_body,
        out_shape=jax.ShapeDtypeStruct((S * B, D // 2, H // 2, W // 2, C), x.dtype),
        grid=(S * B,),
        in_specs=[pl.BlockSpec((1, D, H, W, C), lambda b: (b, 0, 0, 0, 0))],
        out_specs=pl.BlockSpec((1, D // 2, H // 2, W // 2, C),
                               lambda b: (b, 0, 0, 0, 0)),
        compiler_params=pltpu.CompilerParams(
            dimension_semantics=("parallel",),
            vmem_limit_bytes=_VMEM_LIMIT),
    )(xf)
    return out.reshape(S, B, D // 2, H // 2, W // 2, C)


def _avgpool_body(x_ref, o_ref):
    o_ref[...] = jnp.mean(x_ref[...].astype(jnp.float32), axis=1)


def _global_avgpool(x):
    S, B, D, H, W, C = x.shape
    M = S * B
    xf = x.reshape(M, D * H * W, C)
    out = pl.pallas_call(
        _avgpool_body,
        out_shape=jax.ShapeDtypeStruct((M, C), jnp.float32),
        grid=(1,),
        in_specs=[pl.BlockSpec((M, D * H * W, C), lambda i: (0, 0, 0))],
        out_specs=pl.BlockSpec((M, C), lambda i: (0, 0)),
        compiler_params=pltpu.CompilerParams(
            vmem_limit_bytes=_VMEM_LIMIT),
    )(xf)
    return out.reshape(S, B, C)


def _basic_block(x, conv1_w, bn1, conv2_w, bn2, down, *, stride):
    out = _conv_bn_act(x, conv1_w, *bn1, ksize=(3, 3, 3), stride=stride,
                       padding=1, relu=True)
    if down is None:
        residual = x
    else:
        dw, ds, dsh = down
        xs = x[:, :, ::stride, ::stride, ::stride, :]
        S, B, Do, Ho, Wo, C = xs.shape
        a = xs.reshape(S, B * Do * Ho * Wo, C)
        residual = _gemm_bn(a, dw, ds, dsh, relu=False).reshape(
            S, B, Do, Ho, Wo, dw.shape[-1])
    out = _conv_bn_act(out, conv2_w, *bn2, ksize=(3, 3, 3), stride=1,
                       padding=1, relu=True, res=residual)
    return out


def _l2n(v, axis=-1, eps=1e-12):
    n = jnp.sqrt(jnp.sum(v * v, axis=axis, keepdims=True))
    return v / jnp.maximum(n, eps)


def _stack_w(wm, wp):
    w = jnp.stack([wm, wp])
    S = 2
    return w.reshape(S, -1, w.shape[-1])


def _stack_bn(sm, shm, sp, shp):
    return jnp.stack([sm, sp]), jnp.stack([shm, shp])


def kernel(mri_conv1_w, mri_bn1_scale, mri_bn1_shift, mri_l1_conv1_w, mri_l1_bn1_scale, mri_l1_bn1_shift, mri_l1_conv2_w, mri_l1_bn2_scale, mri_l1_bn2_shift, mri_l2_conv1_w, mri_l2_bn1_scale, mri_l2_bn1_shift, mri_l2_conv2_w, mri_l2_bn2_scale, mri_l2_bn2_shift, mri_l2_down_w, mri_l2_down_scale, mri_l2_down_shift, mri_l3_conv1_w, mri_l3_bn1_scale, mri_l3_bn1_shift, mri_l3_conv2_w, mri_l3_bn2_scale, mri_l3_bn2_shift, mri_l3_down_w, mri_l3_down_scale, mri_l3_down_shift, mri_l4_conv1_w, mri_l4_bn1_scale, mri_l4_bn1_shift, mri_l4_conv2_w, mri_l4_bn2_scale, mri_l4_bn2_shift, mri_l4_down_w, mri_l4_down_scale, mri_l4_down_shift, pet_conv1_w, pet_bn1_scale, pet_bn1_shift, pet_l1_conv1_w, pet_l1_bn1_scale, pet_l1_bn1_shift, pet_l1_conv2_w, pet_l1_bn2_scale, pet_l1_bn2_shift, pet_l2_conv1_w, pet_l2_bn1_scale, pet_l2_bn1_shift, pet_l2_conv2_w, pet_l2_bn2_scale, pet_l2_bn2_shift, pet_l2_down_w, pet_l2_down_scale, pet_l2_down_shift, pet_l3_conv1_w, pet_l3_bn1_scale, pet_l3_bn1_shift, pet_l3_conv2_w, pet_l3_bn2_scale, pet_l3_bn2_shift, pet_l3_down_w, pet_l3_down_scale, pet_l3_down_shift, pet_l4_conv1_w, pet_l4_bn1_scale, pet_l4_bn1_shift, pet_l4_conv2_w, pet_l4_bn2_scale, pet_l4_bn2_shift, pet_l4_down_w, pet_l4_down_scale, pet_l4_down_shift, mri_proj_w, mri_proj_shift, pet_proj_w, pet_proj_shift, mri_proj_scale, pet_proj_scale, fc_w, fc_scale, fc_shift, t, mri, pet):
    x = jnp.stack([mri, pet])
    x = jnp.transpose(x, (0, 1, 3, 4, 5, 2)).astype(jnp.bfloat16)

    conv1_w = _stack_w(mri_conv1_w, pet_conv1_w)
    bn1 = _stack_bn(mri_bn1_scale, mri_bn1_shift, pet_bn1_scale, pet_bn1_shift)

    x = _conv_bn_act(x, conv1_w, *bn1, ksize=(7, 7, 7), stride=2, padding=3,
                     relu=True)
    x = _maxpool3d(x)

    layer_params = [
        ((mri_l1_conv1_w, pet_l1_conv1_w),
         (mri_l1_bn1_scale, mri_l1_bn1_shift, pet_l1_bn1_scale, pet_l1_bn1_shift),
         (mri_l1_conv2_w, pet_l1_conv2_w),
         (mri_l1_bn2_scale, mri_l1_bn2_shift, pet_l1_bn2_scale, pet_l1_bn2_shift),
         None, 1),
        ((mri_l2_conv1_w, pet_l2_conv1_w),
         (mri_l2_bn1_scale, mri_l2_bn1_shift, pet_l2_bn1_scale, pet_l2_bn1_shift),
         (mri_l2_conv2_w, pet_l2_conv2_w),
         (mri_l2_bn2_scale, mri_l2_bn2_shift, pet_l2_bn2_scale, pet_l2_bn2_shift),
         ((mri_l2_down_w, pet_l2_down_w),
          (mri_l2_down_scale, mri_l2_down_shift, pet_l2_down_scale, pet_l2_down_shift)), 2),
        ((mri_l3_conv1_w, pet_l3_conv1_w),
         (mri_l3_bn1_scale, mri_l3_bn1_shift, pet_l3_bn1_scale, pet_l3_bn1_shift),
         (mri_l3_conv2_w, pet_l3_conv2_w),
         (mri_l3_bn2_scale, mri_l3_bn2_shift, pet_l3_bn2_scale, pet_l3_bn2_shift),
         ((mri_l3_down_w, pet_l3_down_w),
          (mri_l3_down_scale, mri_l3_down_shift, pet_l3_down_scale, pet_l3_down_shift)), 2),
        ((mri_l4_conv1_w, pet_l4_conv1_w),
         (mri_l4_bn1_scale, mri_l4_bn1_shift, pet_l4_bn1_scale, pet_l4_bn1_shift),
         (mri_l4_conv2_w, pet_l4_conv2_w),
         (mri_l4_bn2_scale, mri_l4_bn2_shift, pet_l4_bn2_scale, pet_l4_bn2_shift),
         ((mri_l4_down_w, pet_l4_down_w),
          (mri_l4_down_scale, mri_l4_down_shift, pet_l4_down_scale, pet_l4_down_shift)), 2),
    ]

    for c1, b1, c2, b2, down, stride in layer_params:
        c1w = _stack_w(*c1)
        c2w = _stack_w(*c2)
        bn_a = _stack_bn(*b1)
        bn_b = _stack_bn(*b2)
        if down is None:
            dn = None
        else:
            (dwm, dwp), (dsm, dshm, dsp, dshp) = down
            dn = (_stack_w(dwm, dwp),) + _stack_bn(dsm, dshm, dsp, dshp)
        x = _basic_block(x, c1w, bn_a, c2w, bn_b, dn, stride=stride)

    feats = _global_avgpool(x)
    mri_feat, pet_feat = feats[0], feats[1]

    proj_w = jnp.concatenate([mri_proj_w, pet_proj_w], axis=0)
    proj_scale = jnp.stack([mri_proj_scale, pet_proj_scale])
    proj_shift = jnp.stack([mri_proj_shift, pet_proj_shift])
    proj = _gemm_bn(feats.astype(jnp.bfloat16), proj_w, proj_scale, proj_shift,
                    relu=False, out_dtype=jnp.float32)

    mri_emb = _l2n(proj[0])
    pet_emb = _l2n(proj[1])
    sim = (mri_emb @ pet_emb.T) / t
    B = sim.shape[0]
    targets = jnp.eye(B, dtype=sim.dtype)
    loss_m2p = -jnp.mean(jnp.sum(jax.nn.log_softmax(sim, axis=1) * targets, axis=1))
    loss_p2m = -jnp.mean(jnp.sum(jax.nn.log_softmax(sim, axis=0) * targets, axis=0))
    loss_cl = 0.5 * (loss_m2p + loss_p2m)

    cat_feat = _l2n(jnp.concatenate([mri_feat, pet_feat], axis=1), axis=1)
    logits = _gemm_bn(cat_feat.astype(jnp.bfloat16)[None], fc_w,
                      fc_scale[None], fc_shift[None], relu=False,
                      out_dtype=jnp.float32)[0]
    return logits, loss_cl


# speedup vs baseline: 179.0661x; 27.1048x over previous
import functools

import jax
import jax.numpy as jnp
from jax.experimental import pallas as pl
from jax.experimental.pallas import tpu as pltpu

_VMEM_LIMIT = 60 * 1024 * 1024


def _s2d(x):
    S, B, D, H, W, C = x.shape
    x = x.reshape(S, B, D // 2, 2, H // 2, 2, W // 2, 2, C)
    x = jnp.transpose(x, (0, 1, 2, 4, 6, 3, 5, 7, 8))
    return x.reshape(S, B, D // 2, H // 2, W // 2, 8 * C)


def _pad_spatial(x, lo, hi):
    cfg = ((0, 0), (0, 0), (lo, hi), (lo, hi), (lo, hi), (0, 0))
    return jnp.pad(x, cfg)


def _w_s2(w, k, cin):
    n = w.shape[-1]
    kb = (k + 1) // 2
    w6 = w.reshape(k, k, k, cin, n)
    w6 = jnp.pad(w6, ((1, 0), (1, 0), (1, 0), (0, 0), (0, 0)))
    w6 = w6.reshape(kb, 2, kb, 2, kb, 2, cin, n)
    w6 = jnp.transpose(w6, (0, 2, 4, 1, 3, 5, 6, 7))
    return w6.reshape(kb * kb * kb * 8 * cin, n)


def _stack_w(wm, wp):
    w = jnp.stack([wm, wp])
    return w.reshape(2, -1, w.shape[-1])


def _stack_w_s2(wm, wp, k, cin):
    return jnp.stack([_w_s2(wm, k, cin), _w_s2(wp, k, cin)])


def _stack_bn(sm, shm, sp, shp):
    return jnp.stack([sm, sp]), jnp.stack([shm, shp])


def _conv_pb_body(x_ref, w_ref, s_ref, b_ref, *rest, kb, tdo, ho, wo, relu,
                  has_res):
    if has_res:
        r_ref, o_ref = rest
    else:
        (o_ref,) = rest
    m = pl.program_id(2)
    parts = []
    for i in range(kb):
        for j in range(kb):
            for l in range(kb):
                sl = x_ref[0, 0, pl.ds(m * tdo + i, tdo), j:j + ho,
                           l:l + wo, :]
                parts.append(sl.reshape(tdo * ho * wo, -1))
    a = jnp.concatenate(parts, axis=-1)
    acc = jnp.dot(a, w_ref[0], preferred_element_type=jnp.float32)
    out = acc * s_ref[0] + b_ref[0]
    if has_res:
        out = out + r_ref[...].reshape(tdo * ho * wo, -1).astype(jnp.float32)
    if relu:
        out = jnp.maximum(out, 0.0)
    n = out.shape[-1]
    o_ref[...] = out.astype(o_ref.dtype).reshape(1, 1, tdo, ho, wo, n)


def _conv_pb(x, w, scale, shift, *, kb, tdo, relu, res=None,
             out_dtype=jnp.bfloat16):
    S, B, Dp, Hp, Wp, C = x.shape
    Do, Ho, Wo = Dp - kb + 1, Hp - kb + 1, Wp - kb + 1
    N = w.shape[-1]
    grid = (S, B, Do // tdo)
    in_specs = [
        pl.BlockSpec((1, 1, Dp, Hp, Wp, C), lambda s, b, m: (s, b, 0, 0, 0, 0)),
        pl.BlockSpec((1, w.shape[1], N), lambda s, b, m: (s, 0, 0)),
        pl.BlockSpec((1, 1, N), lambda s, b, m: (s, 0, 0)),
        pl.BlockSpec((1, 1, N), lambda s, b, m: (s, 0, 0)),
    ]
    args = [x, w, scale, shift]
    if res is not None:
        in_specs.append(pl.BlockSpec((1, 1, tdo, Ho, Wo, N),
                                     lambda s, b, m: (s, b, m, 0, 0, 0)))
        args.append(res)
    body = functools.partial(_conv_pb_body, kb=kb, tdo=tdo, ho=Ho, wo=Wo,
                             relu=relu, has_res=res is not None)
    return pl.pallas_call(
        body,
        out_shape=jax.ShapeDtypeStruct((S, B, Do, Ho, Wo, N), out_dtype),
        grid=grid,
        in_specs=in_specs,
        out_specs=pl.BlockSpec((1, 1, tdo, Ho, Wo, N),
                               lambda s, b, m: (s, b, m, 0, 0, 0)),
        compiler_params=pltpu.CompilerParams(
            dimension_semantics=("parallel", "parallel", "parallel"),
            vmem_limit_bytes=_VMEM_LIMIT),
    )(*args)


def _conv_wb_body(x_ref, w_ref, s_ref, b_ref, *rest, kb, nb, do, ho, wo, relu,
                  has_res):
    if has_res:
        r_ref, o_ref = rest
    else:
        (o_ref,) = rest
    rows = []
    for b in range(nb):
        parts = []
        for i in range(kb):
            for j in range(kb):
                for l in range(kb):
                    sl = x_ref[0, b, i:i + do, j:j + ho, l:l + wo, :]
                    parts.append(sl.reshape(do * ho * wo, -1))
        rows.append(jnp.concatenate(parts, axis=-1))
    a = jnp.concatenate(rows, axis=0)
    acc = jnp.dot(a, w_ref[0], preferred_element_type=jnp.float32)
    out = acc * s_ref[0] + b_ref[0]
    if has_res:
        out = out + r_ref[0].astype(jnp.float32)
    if relu:
        out = jnp.maximum(out, 0.0)
    o_ref[0] = out.astype(o_ref.dtype)


def _conv_wb(x, w, scale, shift, *, kb, relu, res=None,
             out_dtype=jnp.bfloat16):
    S, B, Dp, Hp, Wp, C = x.shape
    Do, Ho, Wo = Dp - kb + 1, Hp - kb + 1, Wp - kb + 1
    N = w.shape[-1]
    M = B * Do * Ho * Wo
    in_specs = [
        pl.BlockSpec((1, B, Dp, Hp, Wp, C), lambda s: (s, 0, 0, 0, 0, 0)),
        pl.BlockSpec((1, w.shape[1], N), lambda s: (s, 0, 0)),
        pl.BlockSpec((1, 1, N), lambda s: (s, 0, 0)),
        pl.BlockSpec((1, 1, N), lambda s: (s, 0, 0)),
    ]
    args = [x, w, scale, shift]
    if res is not None:
        in_specs.append(pl.BlockSpec((1, M, N), lambda s: (s, 0, 0)))
        args.append(res)
    body = functools.partial(_conv_wb_body, kb=kb, nb=B, do=Do, ho=Ho, wo=Wo,
                             relu=relu, has_res=res is not None)
    return pl.pallas_call(
        body,
        out_shape=jax.ShapeDtypeStruct((S, M, N), out_dtype),
        grid=(S,),
        in_specs=in_specs,
        out_specs=pl.BlockSpec((1, M, N), lambda s: (s, 0, 0)),
        compiler_params=pltpu.CompilerParams(
            dimension_semantics=("parallel",),
            vmem_limit_bytes=_VMEM_LIMIT),
    )(*args)


def _gemm_body(a_ref, w_ref, s_ref, b_ref, o_ref, *, relu):
    acc = jnp.dot(a_ref[0], w_ref[0], preferred_element_type=jnp.float32)
    out = acc * s_ref[0] + b_ref[0]
    if relu:
        out = jnp.maximum(out, 0.0)
    o_ref[0] = out.astype(o_ref.dtype)


def _gemm_bn(a, w, scale, shift, *, relu, out_dtype=jnp.bfloat16, tm=4096):
    S, M, K = a.shape
    N = w.shape[-1]
    TM = min(M, tm)
    grid = (S, pl.cdiv(M, TM))
    return pl.pallas_call(
        functools.partial(_gemm_body, relu=relu),
        out_shape=jax.ShapeDtypeStruct((S, M, N), out_dtype),
        grid=grid,
        in_specs=[
            pl.BlockSpec((1, TM, K), lambda s, m: (s, m, 0)),
            pl.BlockSpec((1, K, N), lambda s, m: (s, 0, 0)),
            pl.BlockSpec((1, 1, N), lambda s, m: (s, 0, 0)),
            pl.BlockSpec((1, 1, N), lambda s, m: (s, 0, 0)),
        ],
        out_specs=pl.BlockSpec((1, TM, N), lambda s, m: (s, m, 0)),
        compiler_params=pltpu.CompilerParams(
            dimension_semantics=("parallel", "parallel"),
            vmem_limit_bytes=_VMEM_LIMIT),
    )(a, w, scale, shift)


def _axis_pool(x, axis):
    L = x.shape[axis]
    Lo = L // 2
    shp = x.shape[:axis] + (Lo, 2) + x.shape[axis + 1:]
    xr = x.reshape(shp)
    even = jax.lax.index_in_dim(xr, 0, axis + 1, keepdims=False)
    odd = jax.lax.index_in_dim(xr, 1, axis + 1, keepdims=False)
    pairmax = jnp.maximum(even, odd)
    neg = jnp.finfo(x.dtype).min
    head = jnp.full(odd.shape[:axis] + (1,) + odd.shape[axis + 1:], neg,
                    x.dtype)
    prev_odd = jnp.concatenate(
        [head, jax.lax.slice_in_dim(odd, 0, Lo - 1, axis=axis)], axis=axis)
    return jnp.maximum(pairmax, prev_odd)


def _maxpool_body(x_ref, o_ref):
    x = x_ref[0]
    x = _axis_pool(x, 2)
    x = _axis_pool(x, 1)
    x = _axis_pool(x, 0)
    o_ref[0] = x


def _maxpool3d(x):
    S, B, D, H, W, C = x.shape
    xf = x.reshape(S * B, D, H, W, C)
    out = pl.pallas_call(
        _maxpool_body,
        out_shape=jax.ShapeDtypeStruct((S * B, D // 2, H // 2, W // 2, C),
                                       x.dtype),
        grid=(S * B,),
        in_specs=[pl.BlockSpec((1, D, H, W, C), lambda b: (b, 0, 0, 0, 0))],
        out_specs=pl.BlockSpec((1, D // 2, H // 2, W // 2, C),
                               lambda b: (b, 0, 0, 0, 0)),
        compiler_params=pltpu.CompilerParams(
            dimension_semantics=("parallel",),
            vmem_limit_bytes=_VMEM_LIMIT),
    )(xf)
    return out.reshape(S, B, D // 2, H // 2, W // 2, C)


def _avgpool_body(x_ref, o_ref):
    o_ref[...] = jnp.mean(x_ref[...].astype(jnp.float32), axis=1)


def _global_avgpool(x):
    S, B, D, H, W, C = x.shape
    M = S * B
    xf = x.reshape(M, D * H * W, C)
    out = pl.pallas_call(
        _avgpool_body,
        out_shape=jax.ShapeDtypeStruct((M, C), jnp.float32),
        grid=(1,),
        in_specs=[pl.BlockSpec((M, D * H * W, C), lambda i: (0, 0, 0))],
        out_specs=pl.BlockSpec((M, C), lambda i: (0, 0)),
        compiler_params=pltpu.CompilerParams(vmem_limit_bytes=_VMEM_LIMIT),
    )(xf)
    return out.reshape(S, B, C)


def _l2n(v, axis=-1, eps=1e-12):
    n = jnp.sqrt(jnp.sum(v * v, axis=axis, keepdims=True))
    return v / jnp.maximum(n, eps)


def kernel(mri_conv1_w, mri_bn1_scale, mri_bn1_shift, mri_l1_conv1_w, mri_l1_bn1_scale, mri_l1_bn1_shift, mri_l1_conv2_w, mri_l1_bn2_scale, mri_l1_bn2_shift, mri_l2_conv1_w, mri_l2_bn1_scale, mri_l2_bn1_shift, mri_l2_conv2_w, mri_l2_bn2_scale, mri_l2_bn2_shift, mri_l2_down_w, mri_l2_down_scale, mri_l2_down_shift, mri_l3_conv1_w, mri_l3_bn1_scale, mri_l3_bn1_shift, mri_l3_conv2_w, mri_l3_bn2_scale, mri_l3_bn2_shift, mri_l3_down_w, mri_l3_down_scale, mri_l3_down_shift, mri_l4_conv1_w, mri_l4_bn1_scale, mri_l4_bn1_shift, mri_l4_conv2_w, mri_l4_bn2_scale, mri_l4_bn2_shift, mri_l4_down_w, mri_l4_down_scale, mri_l4_down_shift, pet_conv1_w, pet_bn1_scale, pet_bn1_shift, pet_l1_conv1_w, pet_l1_bn1_scale, pet_l1_bn1_shift, pet_l1_conv2_w, pet_l1_bn2_scale, pet_l1_bn2_shift, pet_l2_conv1_w, pet_l2_bn1_scale, pet_l2_bn1_shift, pet_l2_conv2_w, pet_l2_bn2_scale, pet_l2_bn2_shift, pet_l2_down_w, pet_l2_down_scale, pet_l2_down_shift, pet_l3_conv1_w, pet_l3_bn1_scale, pet_l3_bn1_shift, pet_l3_conv2_w, pet_l3_bn2_scale, pet_l3_bn2_shift, pet_l3_down_w, pet_l3_down_scale, pet_l3_down_shift, pet_l4_conv1_w, pet_l4_bn1_scale, pet_l4_bn1_shift, pet_l4_conv2_w, pet_l4_bn2_scale, pet_l4_bn2_shift, pet_l4_down_w, pet_l4_down_scale, pet_l4_down_shift, mri_proj_w, mri_proj_shift, pet_proj_w, pet_proj_shift, mri_proj_scale, pet_proj_scale, fc_w, fc_scale, fc_shift, t, mri, pet):
    B = mri.shape[0]

    x = jnp.stack([mri, pet])
    x = x.reshape(2, B, 3, 32, 2, 32, 2, 32, 2)
    x = jnp.transpose(x, (0, 1, 3, 5, 7, 4, 6, 8, 2)).astype(jnp.bfloat16)
    x = x.reshape(2, B, 32, 32, 32, 24)
    x = _pad_spatial(x, 2, 1)

    conv1_w = _stack_w_s2(mri_conv1_w, pet_conv1_w, 7, 3)
    bn1 = _stack_bn(mri_bn1_scale, mri_bn1_shift, pet_bn1_scale, pet_bn1_shift)
    x = _conv_pb(x, conv1_w, *bn1, kb=4, tdo=4, relu=True)

    x = _maxpool3d(x)

    w11 = _stack_w(mri_l1_conv1_w, pet_l1_conv1_w)
    w12 = _stack_w(mri_l1_conv2_w, pet_l1_conv2_w)
    b11 = _stack_bn(mri_l1_bn1_scale, mri_l1_bn1_shift,
                    pet_l1_bn1_scale, pet_l1_bn1_shift)
    b12 = _stack_bn(mri_l1_bn2_scale, mri_l1_bn2_shift,
                    pet_l1_bn2_scale, pet_l1_bn2_shift)
    y = _conv_pb(_pad_spatial(x, 1, 1), w11, *b11, kb=3, tdo=8, relu=True)
    x = _conv_pb(_pad_spatial(y, 1, 1), w12, *b12, kb=3, tdo=8, relu=True,
                 res=x)

    w21 = _stack_w_s2(mri_l2_conv1_w, pet_l2_conv1_w, 3, 64)
    w22 = _stack_w(mri_l2_conv2_w, pet_l2_conv2_w)
    b21 = _stack_bn(mri_l2_bn1_scale, mri_l2_bn1_shift,
                    pet_l2_bn1_scale, pet_l2_bn1_shift)
    b22 = _stack_bn(mri_l2_bn2_scale, mri_l2_bn2_shift,
                    pet_l2_bn2_scale, pet_l2_bn2_shift)
    dw2 = _stack_w(mri_l2_down_w, pet_l2_down_w)
    db2 = _stack_bn(mri_l2_down_scale, mri_l2_down_shift,
                    pet_l2_down_scale, pet_l2_down_shift)
    xs = _s2d(x)
    y = _conv_pb(_pad_spatial(xs, 1, 0), w21, *b21, kb=2, tdo=8, relu=True)
    xd = x[:, :, ::2, ::2, ::2, :].reshape(2, -1, 64)
    r = _gemm_bn(xd, dw2, *db2, relu=False).reshape(2, B, 8, 8, 8, 128)
    x = _conv_pb(_pad_spatial(y, 1, 1), w22, *b22, kb=3, tdo=8, relu=True,
                 res=r)

    w31 = _stack_w_s2(mri_l3_conv1_w, pet_l3_conv1_w, 3, 128)
    w32 = _stack_w(mri_l3_conv2_w, pet_l3_conv2_w)
    b31 = _stack_bn(mri_l3_bn1_scale, mri_l3_bn1_shift,
                    pet_l3_bn1_scale, pet_l3_bn1_shift)
    b32 = _stack_bn(mri_l3_bn2_scale, mri_l3_bn2_shift,
                    pet_l3_bn2_scale, pet_l3_bn2_shift)
    dw3 = _stack_w(mri_l3_down_w, pet_l3_down_w)
    db3 = _stack_bn(mri_l3_down_scale, mri_l3_down_shift,
                    pet_l3_down_scale, pet_l3_down_shift)
    xs = _s2d(x)
    y = _conv_wb(_pad_spatial(xs, 1, 0), w31, *b31, kb=2, relu=True)
    y = y.reshape(2, B, 4, 4, 4, 256)
    xd = x[:, :, ::2, ::2, ::2, :].reshape(2, -1, 128)
    r = _gemm_bn(xd, dw3, *db3, relu=False)
    x = _conv_wb(_pad_spatial(y, 1, 1), w32, *b32, kb=3, relu=True, res=r)
    x = x.reshape(2, B, 4, 4, 4, 256)

    w41 = _stack_w_s2(mri_l4_conv1_w, pet_l4_conv1_w, 3, 256)
    w42 = _stack_w(mri_l4_conv2_w, pet_l4_conv2_w)
    b41 = _stack_bn(mri_l4_bn1_scale, mri_l4_bn1_shift,
                    pet_l4_bn1_scale, pet_l4_bn1_shift)
    b42 = _stack_bn(mri_l4_bn2_scale, mri_l4_bn2_shift,
                    pet_l4_bn2_scale, pet_l4_bn2_shift)
    dw4 = _stack_w(mri_l4_down_w, pet_l4_down_w)
    db4 = _stack_bn(mri_l4_down_scale, mri_l4_down_shift,
                    pet_l4_down_scale, pet_l4_down_shift)
    xs = _s2d(x)
    y = _conv_wb(_pad_spatial(xs, 1, 0), w41, *b41, kb=2, relu=True)
    y = y.reshape(2, B, 2, 2, 2, 512)
    xd = x[:, :, ::2, ::2, ::2, :].reshape(2, -1, 256)
    r = _gemm_bn(xd, dw4, *db4, relu=False)
    x = _conv_wb(_pad_spatial(y, 1, 1), w42, *b42, kb=3, relu=True, res=r)
    x = x.reshape(2, B, 2, 2, 2, 512)

    feats = _global_avgpool(x)
    mri_feat, pet_feat = feats[0], feats[1]

    proj_w = jnp.concatenate([mri_proj_w, pet_proj_w], axis=0)
    proj_scale = jnp.stack([mri_proj_scale, pet_proj_scale])
    proj_shift = jnp.stack([mri_proj_shift, pet_proj_shift])
    proj = _gemm_bn(feats.astype(jnp.bfloat16), proj_w, proj_scale,
                    proj_shift, relu=False, out_dtype=jnp.float32)

    mri_emb = _l2n(proj[0])
    pet_emb = _l2n(proj[1])
    sim = (mri_emb @ pet_emb.T) / t
    targets = jnp.eye(B, dtype=sim.dtype)
    loss_m2p = -jnp.mean(jnp.sum(jax.nn.log_softmax(sim, axis=1) * targets,
                                 axis=1))
    loss_p2m = -jnp.mean(jnp.sum(jax.nn.log_softmax(sim, axis=0) * targets,
                                 axis=0))
    loss_cl = 0.5 * (loss_m2p + loss_p2m)

    cat_feat = _l2n(jnp.concatenate([mri_feat, pet_feat], axis=1), axis=1)
    logits = _gemm_bn(cat_feat.astype(jnp.bfloat16)[None], fc_w,
                      fc_scale[None], fc_shift[None], relu=False,
                      out_dtype=jnp.float32)[0]
    return logits, loss_cl
